# Initial kernel scaffold; baseline (speedup 1.0000x reference)
import jax
import jax.numpy as jnp
from jax.experimental import pallas as pl


def kernel(q_len, k_len, row_bias_table, col_bias_table):
    raise NotImplementedError("write your pallas kernel here")


# speedup vs baseline: 1.0092x; 1.0092x over previous
import functools

import jax
import jax.numpy as jnp
from jax import lax
from jax.experimental import pallas as pl
from jax.experimental.pallas import tpu as pltpu
from jax.experimental.pallas import tpu_sc as plsc

_H = 32
_N = 64
_PLANE = _N * _N


def _bias_body(rt_hbm, ct_hbm, out_hbm, rt_v, ct_v, out_v):
    nc = plsc.get_sparse_core_info().num_cores
    wid = lax.axis_index("s") * nc + lax.axis_index("c")

    pltpu.sync_copy(rt_hbm.at[wid], rt_v)
    pltpu.sync_copy(ct_hbm.at[wid], ct_v)

    lane = lax.broadcasted_iota(jnp.int32, (16,), 0)

    def row_body(i, _):
        xbase = i // 8 + 7
        ybase = i % 8 + 7
        for c in range(4):
            j = c * 16 + lane
            xi = xbase - (j >> 3)
            yi = ybase - (j & 7)
            r = plsc.load_gather(rt_v, [xi])
            col = plsc.load_gather(ct_v, [yi])
            out_v[pl.ds(i * _N + c * 16, 16)] = r + col
        return 0

    lax.fori_loop(0, _N, row_body, 0)
    pltpu.sync_copy(out_v, out_hbm.at[wid])


@jax.jit
def _bias_planes(rt_pad, ct_pad):
    mesh = plsc.VectorSubcoreMesh(core_axis_name="c", subcore_axis_name="s")
    return pl.kernel(
        _bias_body,
        mesh=mesh,
        out_type=jax.ShapeDtypeStruct((_H, _PLANE), jnp.float32),
        scratch_types=[
            pltpu.VMEM((16,), jnp.float32),
            pltpu.VMEM((16,), jnp.float32),
            pltpu.VMEM((_PLANE,), jnp.float32),
        ],
        compiler_params=pltpu.CompilerParams(needs_layout_passes=False),
    )(rt_pad, ct_pad)


def kernel(q_len, k_len, row_bias_table, col_bias_table):
    rt_pad = jnp.pad(row_bias_table.T, ((0, 0), (0, 1)))
    ct_pad = jnp.pad(col_bias_table.T, ((0, 0), (0, 1)))
    planes = _bias_planes(rt_pad, ct_pad)
    return planes.reshape(1, _H, _N, _N)


# speedup vs baseline: 1.0457x; 1.0361x over previous
import jax
import jax.numpy as jnp
from jax import lax
from jax.experimental import pallas as pl
from jax.experimental.pallas import tpu as pltpu
from jax.experimental.pallas import tpu_sc as plsc

_H = 32
_N = 64


def _bias_body(rt_hbm, ct_hbm, out_hbm, rt_tab, ct_tab, rt_v, ct_v, out_v):
    nc = plsc.get_sparse_core_info().num_cores
    wid = lax.axis_index("s") * nc + lax.axis_index("c")

    pltpu.sync_copy(rt_hbm, rt_tab)
    pltpu.sync_copy(ct_hbm, ct_tab)
    lane = lax.broadcasted_iota(jnp.int32, (16,), 0)
    rowsel = jnp.minimum(lane, 14)
    colsel = jnp.full((16,), 0, jnp.int32) + wid
    rt_v[...] = plsc.load_gather(rt_tab, [rowsel, colsel])
    ct_v[...] = plsc.load_gather(ct_tab, [rowsel, colsel])

    cvec = [plsc.load_gather(ct_v, [p + 7 - (lane & 7)]) for p in range(8)]

    for a in range(8):
        rvec = [
            plsc.load_gather(rt_v, [a + 7 - ((c * 16 + lane) >> 3)])
            for c in range(4)
        ]
        for p in range(8):
            for c in range(4):
                out_v[a * 8 + p, pl.ds(c * 16, 16)] = rvec[c] + cvec[p]

    pltpu.sync_copy(out_v, out_hbm.at[0, wid])


@jax.jit
def _bias_planes(row_table, col_table):
    mesh = plsc.VectorSubcoreMesh(core_axis_name="c", subcore_axis_name="s")
    return pl.kernel(
        _bias_body,
        mesh=mesh,
        out_type=jax.ShapeDtypeStruct((1, _H, _N, _N), jnp.float32),
        scratch_types=[
            pltpu.VMEM((15, _H), jnp.float32),
            pltpu.VMEM((15, _H), jnp.float32),
            pltpu.VMEM((16,), jnp.float32),
            pltpu.VMEM((16,), jnp.float32),
            pltpu.VMEM((_N, _N), jnp.float32),
        ],
        compiler_params=pltpu.CompilerParams(needs_layout_passes=False),
    )(row_table, col_table)


def kernel(q_len, k_len, row_bias_table, col_bias_table):
    return _bias_planes(row_bias_table, col_bias_table)


# speedup vs baseline: 1.0647x; 1.0182x over previous
import jax
import jax.numpy as jnp
from jax import lax
from jax.experimental import pallas as pl
from jax.experimental.pallas import tpu as pltpu
from jax.experimental.pallas import tpu_sc as plsc

_H = 32
_N = 64


def _bias_body(rt_hbm, ct_hbm, out_hbm, rt_tab, ct_tab, out_v, sem_r, sem_c):
    nc = plsc.get_sparse_core_info().num_cores
    wid = lax.axis_index("s") * nc + lax.axis_index("c")

    cp_r = pltpu.async_copy(rt_hbm, rt_tab, sem_r)
    cp_c = pltpu.async_copy(ct_hbm, ct_tab, sem_c)

    lane = lax.broadcasted_iota(jnp.int32, (16,), 0)
    colsel = jnp.full((16,), 0, jnp.int32) + wid

    cp_c.wait()
    cvec = [plsc.load_gather(ct_tab, [p + 7 - (lane & 7), colsel])
            for p in range(8)]

    cp_r.wait()
    for a in range(8):
        rvec = [
            plsc.load_gather(rt_tab, [a + 7 - ((c * 16 + lane) >> 3), colsel])
            for c in range(4)
        ]
        for p in range(8):
            for c in range(4):
                out_v[a * 8 + p, pl.ds(c * 16, 16)] = rvec[c] + cvec[p]

    pltpu.sync_copy(out_v, out_hbm.at[0, wid])


@jax.jit
def _bias_planes(row_table, col_table):
    mesh = plsc.VectorSubcoreMesh(core_axis_name="c", subcore_axis_name="s")
    return pl.kernel(
        _bias_body,
        mesh=mesh,
        out_type=jax.ShapeDtypeStruct((1, _H, _N, _N), jnp.float32),
        scratch_types=[
            pltpu.VMEM((15, _H), jnp.float32),
            pltpu.VMEM((15, _H), jnp.float32),
            pltpu.VMEM((_N, _N), jnp.float32),
            pltpu.SemaphoreType.DMA,
            pltpu.SemaphoreType.DMA,
        ],
        compiler_params=pltpu.CompilerParams(needs_layout_passes=False),
    )(row_table, col_table)


def kernel(q_len, k_len, row_bias_table, col_bias_table):
    return _bias_planes(row_bias_table, col_bias_table)


# speedup vs baseline: 1.0802x; 1.0146x over previous
import jax
import jax.numpy as jnp
from jax import lax
from jax.experimental import pallas as pl
from jax.experimental.pallas import tpu as pltpu
from jax.experimental.pallas import tpu_sc as plsc

_H = 32
_N = 64


def _bias_body(rt_hbm, ct_hbm, out_hbm, rt_tab, ct_tab, out_v, sem_r, sem_c):
    nc = plsc.get_sparse_core_info().num_cores
    wid = lax.axis_index("s") * nc + lax.axis_index("c")

    cp_r = pltpu.async_copy(rt_hbm, rt_tab, sem_r)
    cp_c = pltpu.async_copy(ct_hbm, ct_tab, sem_c)

    lane = lax.broadcasted_iota(jnp.int32, (16,), 0)
    colsel = jnp.full((16,), 0, jnp.int32) + wid

    cp_c.wait()
    cvec = [plsc.load_gather(ct_tab, [p + 7 - (lane & 7), colsel])
            for p in range(8)]

    cp_r.wait()

    def block_body(a, _):
        rvec = [
            plsc.load_gather(rt_tab, [a + 7 - ((c * 16 + lane) >> 3), colsel])
            for c in range(4)
        ]
        for p in range(8):
            for c in range(4):
                out_v[a * 8 + p, pl.ds(c * 16, 16)] = rvec[c] + cvec[p]
        return 0

    lax.fori_loop(0, 8, block_body, 0)

    pltpu.sync_copy(out_v, out_hbm.at[0, wid])


@jax.jit
def _bias_planes(row_table, col_table):
    mesh = plsc.VectorSubcoreMesh(core_axis_name="c", subcore_axis_name="s")
    return pl.kernel(
        _bias_body,
        mesh=mesh,
        out_type=jax.ShapeDtypeStruct((1, _H, _N, _N), jnp.float32),
        scratch_types=[
            pltpu.VMEM((15, _H), jnp.float32),
            pltpu.VMEM((15, _H), jnp.float32),
            pltpu.VMEM((_N, _N), jnp.float32),
            pltpu.SemaphoreType.DMA,
            pltpu.SemaphoreType.DMA,
        ],
        compiler_params=pltpu.CompilerParams(needs_layout_passes=False),
    )(row_table, col_table)


def kernel(q_len, k_len, row_bias_table, col_bias_table):
    return _bias_planes(row_bias_table, col_bias_table)
